# baseline (speedup 1.0000x reference)
import jax
import jax.numpy as jnp
from jax import lax
from jax.experimental import pallas as pl

N = 10000
D = 128
E = 320000

NPAD = 10240
BLK = 1024
GRID = NPAD // BLK


def _tc1_body(deg, xC, norms, xscc, xscn):
    d = deg[...]
    no_cc = jnp.where(d[0] > 0, lax.rsqrt(d[0]), 0.0)
    ni_cc = jnp.where(d[1] > 0, lax.rsqrt(d[1]), 0.0)
    no_cn = jnp.where(d[2] > 0, lax.rsqrt(d[2]), 0.0)
    ni_cn = jnp.where(d[3] > 0, lax.rsqrt(d[3]), 0.0)
    inv_nn = 1.0 / jnp.maximum(d[4], 1.0)
    z = jnp.zeros_like(no_cc)
    norms[...] = jnp.stack([no_cc, ni_cc, no_cn, ni_cn, inv_nn, z, z, z])
    x = xC[...]
    xscc[...] = x * no_cc[:, None]
    xscn[...] = x * no_cn[:, None]


_tc1 = pl.pallas_call(
    _tc1_body,
    grid=(GRID,),
    in_specs=[
        pl.BlockSpec((5, BLK), lambda i: (0, i)),
        pl.BlockSpec((BLK, D), lambda i: (i, 0)),
    ],
    out_specs=[
        pl.BlockSpec((8, BLK), lambda i: (0, i)),
        pl.BlockSpec((BLK, D), lambda i: (i, 0)),
        pl.BlockSpec((BLK, D), lambda i: (i, 0)),
    ],
    out_shape=[
        jax.ShapeDtypeStruct((8, NPAD), jnp.float32),
        jax.ShapeDtypeStruct((NPAD, D), jnp.float32),
        jax.ShapeDtypeStruct((NPAD, D), jnp.float32),
    ],
)


def _mm(a, w_ref):
    return jnp.dot(a, w_ref[...], preferred_element_type=jnp.float32)


def _layer_body(acc, acn, ann, xN, norms, Wcc, Wcn, Ws, Wn, bcc, bcn, bnn,
                relu, outs):
    nm = norms[...]
    no_cc, ni_cc, no_cn, ni_cn, inv_nn = nm[0], nm[1], nm[2], nm[3], nm[4]
    hC = ni_cc[:, None] * _mm(acc[...], Wcc) + bcc[...][None, :]
    gcn = ni_cn[:, None] * _mm(acn[...], Wcn) + bcn[...][None, :]
    mean = ann[...] * inv_nn[:, None]
    hN = gcn + _mm(xN[...], Ws) + _mm(mean, Wn) + bnn[...][None, :]
    if relu:
        hC = jnp.maximum(hC, 0.0)
        hN = jnp.maximum(hN, 0.0)
        hscc, hscn, hN_out = outs
        hscc[...] = hC * no_cc[:, None]
        hscn[...] = hC * no_cn[:, None]
        hN_out[...] = hN
    else:
        oC, oN = outs
        oC[...] = hC
        oN[...] = hN


def _tc2_body(acc, acn, ann, xN, norms, Wcc, Wcn, Ws, Wn, bcc, bcn, bnn,
              hscc, hscn, hN):
    _layer_body(acc, acn, ann, xN, norms, Wcc, Wcn, Ws, Wn, bcc, bcn, bnn,
                True, (hscc, hscn, hN))


def _tc3_body(acc, acn, ann, xN, norms, Wcc, Wcn, Ws, Wn, bcc, bcn, bnn,
              oC, oN):
    _layer_body(acc, acn, ann, xN, norms, Wcc, Wcn, Ws, Wn, bcc, bcn, bnn,
                False, (oC, oN))


def _layer_call(body, n_out):
    row_spec = pl.BlockSpec((BLK, D), lambda i: (i, 0))
    w_spec = pl.BlockSpec((D, D), lambda i: (0, 0))
    b_spec = pl.BlockSpec((D,), lambda i: (0,))
    return pl.pallas_call(
        body,
        grid=(GRID,),
        in_specs=[row_spec, row_spec, row_spec, row_spec,
                  pl.BlockSpec((8, BLK), lambda i: (0, i)),
                  w_spec, w_spec, w_spec, w_spec, b_spec, b_spec, b_spec],
        out_specs=[row_spec] * n_out,
        out_shape=[jax.ShapeDtypeStruct((NPAD, D), jnp.float32)] * n_out,
    )


_tc2 = _layer_call(_tc2_body, 3)
_tc3 = _layer_call(_tc3_body, 2)


def _segsum3(t0, t1, t2, srcall, dstall):
    tables = jnp.concatenate([t0, t1, t2])
    agg = jnp.zeros((3 * NPAD, D), jnp.float32).at[dstall].add(tables[srcall])
    return agg[:NPAD], agg[NPAD:2 * NPAD], agg[2 * NPAD:]


def kernel(x_C, x_N, edge_cc, edge_cn, edge_nn,
           W1_cc, b1_cc, W1_cn, b1_cn, W1s_nn, W1n_nn, b1_nn,
           W2_cc, b2_cc, W2_cn, b2_cn, W2s_nn, W2n_nn, b2_nn):
    xC_pad = jnp.zeros((NPAD, D), jnp.float32).at[:N].set(x_C)
    xN_pad = jnp.zeros((NPAD, D), jnp.float32).at[:N].set(x_N)
    cc_s, cc_d = edge_cc[0], edge_cc[1]
    cn_s, cn_d = edge_cn[0], edge_cn[1]
    nn_s, nn_d = edge_nn[0], edge_nn[1]

    srcall = jnp.concatenate([cc_s, cn_s + NPAD, nn_s + 2 * NPAD])
    dstall = jnp.concatenate([cc_d, cn_d + NPAD, nn_d + 2 * NPAD])

    didx = jnp.concatenate([cc_s, cc_d + NPAD, cn_s + 2 * NPAD,
                            cn_d + 3 * NPAD, nn_d + 4 * NPAD])
    deg = jnp.zeros((5 * NPAD,), jnp.float32).at[didx].add(1.0)
    deg = deg.reshape(5, NPAD)

    norms, xscc, xscn = _tc1(deg, xC_pad)
    a1cc, a1cn, a1nn = _segsum3(xscc, xscn, xN_pad, srcall, dstall)
    hscc, hscn, hN = _tc2(a1cc, a1cn, a1nn, xN_pad, norms,
                          W1_cc, W1_cn, W1s_nn, W1n_nn, b1_cc, b1_cn, b1_nn)
    a2cc, a2cn, a2nn = _segsum3(hscc, hscn, hN, srcall, dstall)
    oC, oN = _tc3(a2cc, a2cn, a2nn, hN, norms,
                  W2_cc, W2_cn, W2s_nn, W2n_nn, b2_cc, b2_cn, b2_nn)
    return oC[:N], oN[:N]


# speedup vs baseline: 1.3714x; 1.3714x over previous
import jax
import jax.numpy as jnp
from jax import lax
from jax.experimental import pallas as pl

N = 10000
D = 128
E = 320000

NPAD = 10240
BLK = 1024
GRID = NPAD // BLK


def _tc1_body(deg, xC, norms, xscc, xscn):
    d = deg[...]
    no_cc = jnp.where(d[0] > 0, lax.rsqrt(d[0]), 0.0)
    ni_cc = jnp.where(d[1] > 0, lax.rsqrt(d[1]), 0.0)
    no_cn = jnp.where(d[2] > 0, lax.rsqrt(d[2]), 0.0)
    ni_cn = jnp.where(d[3] > 0, lax.rsqrt(d[3]), 0.0)
    inv_nn = 1.0 / jnp.maximum(d[4], 1.0)
    z = jnp.zeros_like(no_cc)
    norms[...] = jnp.stack([no_cc, ni_cc, no_cn, ni_cn, inv_nn, z, z, z])
    x = xC[...]
    xscc[...] = x * no_cc[:, None]
    xscn[...] = x * no_cn[:, None]


_tc1 = pl.pallas_call(
    _tc1_body,
    grid=(GRID,),
    in_specs=[
        pl.BlockSpec((5, BLK), lambda i: (0, i)),
        pl.BlockSpec((BLK, D), lambda i: (i, 0)),
    ],
    out_specs=[
        pl.BlockSpec((8, BLK), lambda i: (0, i)),
        pl.BlockSpec((BLK, D), lambda i: (i, 0)),
        pl.BlockSpec((BLK, D), lambda i: (i, 0)),
    ],
    out_shape=[
        jax.ShapeDtypeStruct((8, NPAD), jnp.float32),
        jax.ShapeDtypeStruct((NPAD, D), jnp.float32),
        jax.ShapeDtypeStruct((NPAD, D), jnp.float32),
    ],
)


def _mm(a, w_ref):
    return jnp.dot(a, w_ref[...], preferred_element_type=jnp.float32)


def _layer_body(acc, acn, ann, xN, norms, Wcc, Wcn, Ws, Wn, bcc, bcn, bnn,
                relu, outs):
    nm = norms[...]
    no_cc, ni_cc, no_cn, ni_cn, inv_nn = nm[0], nm[1], nm[2], nm[3], nm[4]
    hC = ni_cc[:, None] * _mm(acc[...], Wcc) + bcc[...][None, :]
    gcn = ni_cn[:, None] * _mm(acn[...], Wcn) + bcn[...][None, :]
    mean = ann[...] * inv_nn[:, None]
    hN = gcn + _mm(xN[...], Ws) + _mm(mean, Wn) + bnn[...][None, :]
    if relu:
        hC = jnp.maximum(hC, 0.0)
        hN = jnp.maximum(hN, 0.0)
        hscc, hscn, hN_out = outs
        hscc[...] = hC * no_cc[:, None]
        hscn[...] = hC * no_cn[:, None]
        hN_out[...] = hN
    else:
        oC, oN = outs
        oC[...] = hC
        oN[...] = hN


def _tc2_body(acc, acn, ann, xN, norms, Wcc, Wcn, Ws, Wn, bcc, bcn, bnn,
              hscc, hscn, hN):
    _layer_body(acc, acn, ann, xN, norms, Wcc, Wcn, Ws, Wn, bcc, bcn, bnn,
                True, (hscc, hscn, hN))


def _tc3_body(acc, acn, ann, xN, norms, Wcc, Wcn, Ws, Wn, bcc, bcn, bnn,
              oC, oN):
    _layer_body(acc, acn, ann, xN, norms, Wcc, Wcn, Ws, Wn, bcc, bcn, bnn,
                False, (oC, oN))


def _layer_call(body, n_out):
    row_spec = pl.BlockSpec((BLK, D), lambda i: (i, 0))
    w_spec = pl.BlockSpec((D, D), lambda i: (0, 0))
    b_spec = pl.BlockSpec((D,), lambda i: (0,))
    return pl.pallas_call(
        body,
        grid=(GRID,),
        in_specs=[row_spec, row_spec, row_spec, row_spec,
                  pl.BlockSpec((8, BLK), lambda i: (0, i)),
                  w_spec, w_spec, w_spec, w_spec, b_spec, b_spec, b_spec],
        out_specs=[row_spec] * n_out,
        out_shape=[jax.ShapeDtypeStruct((NPAD, D), jnp.float32)] * n_out,
    )


_tc2 = _layer_call(_tc2_body, 3)
_tc3 = _layer_call(_tc3_body, 2)


def _segsum(table, src, dst):
    return jnp.zeros((NPAD, D), jnp.float32).at[dst].add(table[src])


def _bincount(idx):
    return jnp.zeros((NPAD,), jnp.float32).at[idx].add(1.0)


def kernel(x_C, x_N, edge_cc, edge_cn, edge_nn,
           W1_cc, b1_cc, W1_cn, b1_cn, W1s_nn, W1n_nn, b1_nn,
           W2_cc, b2_cc, W2_cn, b2_cn, W2s_nn, W2n_nn, b2_nn):
    xC_pad = jnp.zeros((NPAD, D), jnp.float32).at[:N].set(x_C)
    xN_pad = jnp.zeros((NPAD, D), jnp.float32).at[:N].set(x_N)
    cc_s, cc_d = edge_cc[0], edge_cc[1]
    cn_s, cn_d = edge_cn[0], edge_cn[1]
    nn_s, nn_d = edge_nn[0], edge_nn[1]

    deg = jnp.stack([_bincount(cc_s), _bincount(cc_d), _bincount(cn_s),
                     _bincount(cn_d), _bincount(nn_d)])

    norms, xscc, xscn = _tc1(deg, xC_pad)
    a1cc = _segsum(xscc, cc_s, cc_d)
    a1cn = _segsum(xscn, cn_s, cn_d)
    a1nn = _segsum(xN_pad, nn_s, nn_d)
    hscc, hscn, hN = _tc2(a1cc, a1cn, a1nn, xN_pad, norms,
                          W1_cc, W1_cn, W1s_nn, W1n_nn, b1_cc, b1_cn, b1_nn)
    a2cc = _segsum(hscc, cc_s, cc_d)
    a2cn = _segsum(hscn, cn_s, cn_d)
    a2nn = _segsum(hN, nn_s, nn_d)
    oC, oN = _tc3(a2cc, a2cn, a2nn, hN, norms,
                  W2_cc, W2_cn, W2s_nn, W2n_nn, b2_cc, b2_cn, b2_nn)
    return oC[:N], oN[:N]
